# baseline (speedup 1.0000x reference)
import dataclasses

import jax
import jax.numpy as jnp
from jax import lax
from jax.experimental import pallas as pl
from jax.experimental.pallas import tpu as pltpu
from jax.experimental.pallas import tpu_sc as plsc

N = 10000
D = 128
H = 128
OUT = 128
G = 64
E = 320000

NP = 10112
NC, NS, L = 2, 16, 16
NW = NC * NS
C = 64
K = 160
K0 = 176
K1 = 2 * K - K0
EP = NW * C * K
EPW = K * C
RPS = NP // NS
RB = 4
IB = 8


def _leaky(z):
    return jnp.maximum(z, 0.0) + 0.2 * jnp.minimum(z, 0.0)


def _tc_prep_body_inner(xv, W_ref, asrc_ref, adst_ref,
                        h_ref, selfc_ref, selfd_ref, as_ref, ad_ref):
    h = jnp.dot(xv, W_ref[...], preferred_element_type=jnp.float32)
    a_s = jnp.sum(h * asrc_ref[...], axis=-1)
    a_d = jnp.sum(h * adst_ref[...], axis=-1)
    wself = jnp.exp(_leaky(a_s + a_d))
    h_ref[...] = h
    selfc_ref[...] = h * wself[:, None]
    selfd_ref[...] = wself
    as_ref[...] = a_s
    ad_ref[...] = a_d


def _tc_prep_body(x_ref, W_ref, asrc_ref, adst_ref,
                  h_ref, selfc_ref, selfd_ref, as_ref, ad_ref):
    _tc_prep_body_inner(x_ref[...], W_ref, asrc_ref, adst_ref,
                        h_ref, selfc_ref, selfd_ref, as_ref, ad_ref)


_PREP_OUT = [
    jax.ShapeDtypeStruct((NP, H), jnp.float32),
    jax.ShapeDtypeStruct((NP, H), jnp.float32),
    jax.ShapeDtypeStruct((NP,), jnp.float32),
    jax.ShapeDtypeStruct((NP,), jnp.float32),
    jax.ShapeDtypeStruct((NP,), jnp.float32),
]


def _tc_prep(xp, W, a_src, a_dst):
    return pl.pallas_call(_tc_prep_body, out_shape=_PREP_OUT)(
        xp, W, a_src, a_dst)


def _combine(part_ref, pd0_ref, pd1_ref, selfc_ref, selfd_ref, b_ref):
    tot = part_ref[0] + part_ref[1] + selfc_ref[...]
    den = pd0_ref[...] + pd1_ref[...] + selfd_ref[...]
    return tot / den[:, None] + b_ref[...]


def _tc_mid_body(part_ref, pd0_ref, pd1_ref, selfc_ref, selfd_ref, b_ref,
                 W_ref, asrc_ref, adst_ref,
                 h_ref, selfc2_ref, selfd2_ref, as_ref, ad_ref):
    h1 = jnp.maximum(_combine(part_ref, pd0_ref, pd1_ref, selfc_ref,
                              selfd_ref, b_ref), 0.0)
    _tc_prep_body_inner(h1, W_ref, asrc_ref, adst_ref,
                        h_ref, selfc2_ref, selfd2_ref, as_ref, ad_ref)


def _tc_mid(part, pd0, pd1, selfc, selfd, b, W, a_src, a_dst):
    return pl.pallas_call(_tc_mid_body, out_shape=_PREP_OUT)(
        part, pd0, pd1, selfc, selfd, b, W, a_src, a_dst)


def _tc_final_body(part_ref, pd0_ref, pd1_ref, selfc_ref, selfd_ref, b_ref,
                   batch_ref, linW_ref, linb_ref, out_ref):
    h2 = _combine(part_ref, pd0_ref, pd1_ref, selfc_ref, selfd_ref, b_ref)
    gids = lax.broadcasted_iota(jnp.int32, (G, NP), 0)
    onehot = jnp.where(batch_ref[...][None, :] == gids, 1.0, 0.0)
    sums = jnp.dot(onehot, h2, preferred_element_type=jnp.float32)
    counts = jnp.sum(onehot, axis=1)
    pooled = sums / jnp.maximum(counts, 1.0)[:, None]
    out_ref[...] = (jnp.dot(pooled, linW_ref[...],
                            preferred_element_type=jnp.float32)
                    + linb_ref[...])


def _tc_final(part, pd0, pd1, selfc, selfd, b, batchp, lin_W, lin_b):
    return pl.pallas_call(
        _tc_final_body,
        out_shape=jax.ShapeDtypeStruct((G, OUT), jnp.float32),
    )(part, pd0, pd1, selfc, selfd, b, batchp, lin_W, lin_b)


def _sc_body(h_h, as_h, ad_h, src_h, dst_h, zden_h,
             part_h, partd0_h, partd1_h, *scr):
    a_s_v, a_d_v = scr[0], scr[1]
    rows = list(scr[2:4])
    srcr = list(scr[4:8])
    dstr = list(scr[8:12])
    wr = list(scr[12:16])
    acc_s, den_s = scr[16], scr[17]
    sg = list(scr[18:20])
    ss = list(scr[20:22])
    ssrc = list(scr[22:26])
    sdst = list(scr[26:30])
    sw = list(scr[30:34])
    st0, st1 = scr[34], scr[35]

    c = lax.axis_index("c")
    s = lax.axis_index("s")
    kq = jnp.where(c == 0, K0, K1)
    ebase = jnp.where(c == 0, s * (K0 * C), NS * (K0 * C) + s * (K1 * C))

    rsl = pl.ds(s * RPS, RPS)

    pltpu.async_copy(as_h, a_s_v, st0)
    pltpu.async_copy(ad_h, a_d_v, st1)

    @pl.when(s == 0)
    def _():
        pltpu.sync_copy(zden_h, den_s)

    zv = jnp.zeros((L,), jnp.float32)
    zi = jnp.zeros((L,), jnp.int32)
    for i in range(0, C):
        for j in range(H // L):
            rows[1][i, pl.ds(j * L, L)] = zv
    for t in range(RPS // C):
        pltpu.sync_copy(rows[1], acc_s.at[pl.ds(s * RPS + t * C, C)])
    rem = RPS % C
    if rem:
        pltpu.sync_copy(rows[1].at[pl.ds(0, rem)],
                        acc_s.at[pl.ds(s * RPS + (RPS // C) * C, rem)])

    pltpu.make_async_copy(as_h, a_s_v, st0).wait()
    pltpu.make_async_copy(ad_h, a_d_v, st1).wait()
    plsc.subcore_barrier()

    def _idx_copy(k, sl):
        return pltpu.make_async_copy(
            src_h.at[pl.ds(ebase + k * C, C)], srcr[sl], ssrc[sl])

    def _dst_copy(k, sl):
        return pltpu.make_async_copy(
            dst_h.at[pl.ds(ebase + k * C, C)], dstr[sl], sdst[sl])

    def _gather(sl, b):
        return pltpu.make_async_copy(h_h.at[srcr[sl]], rows[b], sg[b])

    def _scat_rows(b, sl):
        return pltpu.make_async_copy(rows[b], acc_s.at[dstr[sl]], ss[b])

    def _scat_w(sl):
        return pltpu.make_async_copy(wr[sl], den_s.at[dstr[sl]], sw[sl])

    for g in range(0, C, L):
        wr[3][pl.ds(g, L)] = zv
        dstr[3][pl.ds(g, L)] = zi
    pltpu.async_copy(rows[1], acc_s.at[dstr[3]], ss[1], add=True)
    pltpu.async_copy(wr[3], den_s.at[dstr[3]], sw[3], add=True)

    for j in range(3):
        _idx_copy(j, j).start()
        _dst_copy(j, j).start()
    _idx_copy(0, 0).wait()
    _gather(0, 0).start()

    @pl.loop(0, kq, step=4)
    def _phase4(kk):
        for p in range(4):
            k = kk + p
            b = p % 2

            _scat_rows(1 - b, (p - 1) % 4).wait()
            _scat_w((p - 1) % 4).wait()

            kn = jnp.minimum(k + 1, kq - 1)
            _idx_copy(kn, (p + 1) % 4).wait()
            _gather((p + 1) % 4, 1 - b).start()

            kp = jnp.minimum(k + 3, kq - 1)
            _idx_copy(kp, (p + 3) % 4).start()
            _dst_copy(kp, (p + 3) % 4).start()

            _gather(p % 4, b).wait()
            _dst_copy(k, p).wait()
            for g in range(0, C, L):
                si = srcr[p % 4][pl.ds(g, L)]
                di = dstr[p][pl.ds(g, L)]
                zz = (plsc.load_gather(a_s_v, [si])
                      + plsc.load_gather(a_d_v, [di]))
                wr[p][pl.ds(g, L)] = jnp.exp(_leaky(zz))

            @plsc.parallel_loop(0, C, unroll=4)
            def _row(i):
                wv = plsc.load_gather(wr[p], [jnp.full((L,), 0, jnp.int32) + i])
                for j in range(H // L):
                    sl = pl.ds(j * L, L)
                    rows[b][i, sl] = rows[b][i, sl] * wv

            pltpu.async_copy(rows[b], acc_s.at[dstr[p]], ss[b], add=True)
            pltpu.async_copy(wr[p], den_s.at[dstr[p]], sw[p], add=True)

    _scat_rows((K - 1) % 2, (K - 1) % 4).wait()
    _scat_w((K - 1) % 4).wait()
    _gather(0, K % 2).wait()
    _idx_copy(kq - 1, (K + 1) % 4).wait()
    _idx_copy(kq - 1, (K + 2) % 4).wait()
    _dst_copy(kq - 1, K % 4).wait()
    _dst_copy(kq - 1, (K + 1) % 4).wait()
    _dst_copy(kq - 1, (K + 2) % 4).wait()

    plsc.subcore_barrier()
    pltpu.sync_copy(acc_s.at[rsl], part_h.at[c, rsl])

    @pl.when(jnp.logical_and(s == 0, c == 0))
    def _():
        pltpu.sync_copy(den_s, partd0_h)

    @pl.when(jnp.logical_and(s == 0, c == 1))
    def _():
        pltpu.sync_copy(den_s, partd1_h)


def _sc_layer(h, a_s, a_d, srcp, dstp3, zden):
    mesh = plsc.VectorSubcoreMesh(core_axis_name="c", subcore_axis_name="s",
                                  num_cores=NC, num_subcores=NS)
    cp = pltpu.CompilerParams()
    if "needs_layout_passes" in pltpu.CompilerParams.__dataclass_fields__:
        cp = dataclasses.replace(cp, needs_layout_passes=False)
    kern = pl.kernel(
        _sc_body,
        out_type=(
            jax.ShapeDtypeStruct((NC, NP, H), jnp.float32),
            jax.ShapeDtypeStruct((NP,), jnp.float32),
            jax.ShapeDtypeStruct((NP,), jnp.float32),
        ),
        mesh=mesh,
        compiler_params=cp,
        scratch_types=(
            [
                pltpu.VMEM((NP,), jnp.float32),
                pltpu.VMEM((NP,), jnp.float32),
            ]
            + [pltpu.VMEM((C, H), jnp.float32)] * 2
            + [pltpu.VMEM((C,), jnp.int32)] * 8
            + [pltpu.VMEM((C,), jnp.float32)] * 4
            + [
                pltpu.VMEM_SHARED((NP, H), jnp.float32),
                pltpu.VMEM_SHARED((NP,), jnp.float32),
            ]
            + [pltpu.SemaphoreType.DMA] * 18
        ),
    )
    return kern(h, a_s, a_d, srcp, dstp3, zden)


def kernel(x, edge_index, batch, W1, a_src1, a_dst1, b1,
           W2, a_src2, a_dst2, b2, lin_W, lin_b):
    src = edge_index[0].astype(jnp.int32)
    dst = edge_index[1].astype(jnp.int32)
    pad_e = EP - E
    srcp = jnp.concatenate([src, jnp.zeros((pad_e,), jnp.int32)])
    pad_dst = N + jnp.arange(pad_e, dtype=jnp.int32) % (NP - N)
    dstp = jnp.concatenate([dst, pad_dst])
    xp = jnp.pad(x, ((0, NP - N), (0, 0)))
    batchp = jnp.concatenate(
        [batch.astype(jnp.int32), jnp.full((NP - N,), G, jnp.int32)])
    zden = jnp.zeros((NP,), jnp.float32)

    h1, selfc1, selfd1, a_s1, a_d1 = _tc_prep(xp, W1, a_src1, a_dst1)
    part1, pd1a, pd1b = _sc_layer(h1, a_s1, a_d1, srcp, dstp, zden)
    h2, selfc2, selfd2, a_s2, a_d2 = _tc_mid(
        part1, pd1a, pd1b, selfc1, selfd1, b1, W2, a_src2, a_dst2)
    part2, pd2a, pd2b = _sc_layer(h2, a_s2, a_d2, srcp, dstp, zden)
    return _tc_final(part2, pd2a, pd2b, selfc2, selfd2, b2, batchp,
                     lin_W, lin_b)


# speedup vs baseline: 1.0521x; 1.0521x over previous
import dataclasses

import jax
import jax.numpy as jnp
from jax import lax
from jax.experimental import pallas as pl
from jax.experimental.pallas import tpu as pltpu
from jax.experimental.pallas import tpu_sc as plsc

N = 10000
D = 128
H = 128
OUT = 128
G = 64
E = 320000

NP = 10112
NC, NS, L = 2, 16, 16
NW = NC * NS
C = 64
K = 160
K0 = 224
K1 = 2 * K - K0
EP = NW * C * K
EPW = K * C
RPS = NP // NS
RB = 4
IB = 8


def _leaky(z):
    return jnp.maximum(z, 0.0) + 0.2 * jnp.minimum(z, 0.0)


def _tc_prep_body_inner(xv, W_ref, asrc_ref, adst_ref,
                        h_ref, selfc_ref, selfd_ref, as_ref, ad_ref):
    h = jnp.dot(xv, W_ref[...], preferred_element_type=jnp.float32)
    a_s = jnp.sum(h * asrc_ref[...], axis=-1)
    a_d = jnp.sum(h * adst_ref[...], axis=-1)
    wself = jnp.exp(_leaky(a_s + a_d))
    h_ref[...] = h
    selfc_ref[...] = h * wself[:, None]
    selfd_ref[...] = wself
    as_ref[...] = a_s
    ad_ref[...] = a_d


def _tc_prep_body(x_ref, W_ref, asrc_ref, adst_ref,
                  h_ref, selfc_ref, selfd_ref, as_ref, ad_ref):
    _tc_prep_body_inner(x_ref[...], W_ref, asrc_ref, adst_ref,
                        h_ref, selfc_ref, selfd_ref, as_ref, ad_ref)


_PREP_OUT = [
    jax.ShapeDtypeStruct((NP, H), jnp.float32),
    jax.ShapeDtypeStruct((NP, H), jnp.float32),
    jax.ShapeDtypeStruct((NP,), jnp.float32),
    jax.ShapeDtypeStruct((NP,), jnp.float32),
    jax.ShapeDtypeStruct((NP,), jnp.float32),
]


def _tc_prep(xp, W, a_src, a_dst):
    return pl.pallas_call(_tc_prep_body, out_shape=_PREP_OUT)(
        xp, W, a_src, a_dst)


def _combine(part_ref, pd0_ref, pd1_ref, selfc_ref, selfd_ref, b_ref):
    tot = part_ref[0] + part_ref[1] + selfc_ref[...]
    den = pd0_ref[...] + pd1_ref[...] + selfd_ref[...]
    return tot / den[:, None] + b_ref[...]


def _tc_mid_body(part_ref, pd0_ref, pd1_ref, selfc_ref, selfd_ref, b_ref,
                 W_ref, asrc_ref, adst_ref,
                 h_ref, selfc2_ref, selfd2_ref, as_ref, ad_ref):
    h1 = jnp.maximum(_combine(part_ref, pd0_ref, pd1_ref, selfc_ref,
                              selfd_ref, b_ref), 0.0)
    _tc_prep_body_inner(h1, W_ref, asrc_ref, adst_ref,
                        h_ref, selfc2_ref, selfd2_ref, as_ref, ad_ref)


def _tc_mid(part, pd0, pd1, selfc, selfd, b, W, a_src, a_dst):
    return pl.pallas_call(_tc_mid_body, out_shape=_PREP_OUT)(
        part, pd0, pd1, selfc, selfd, b, W, a_src, a_dst)


def _tc_final_body(part_ref, pd0_ref, pd1_ref, selfc_ref, selfd_ref, b_ref,
                   batch_ref, linW_ref, linb_ref, out_ref):
    h2 = _combine(part_ref, pd0_ref, pd1_ref, selfc_ref, selfd_ref, b_ref)
    gids = lax.broadcasted_iota(jnp.int32, (G, NP), 0)
    onehot = jnp.where(batch_ref[...][None, :] == gids, 1.0, 0.0)
    sums = jnp.dot(onehot, h2, preferred_element_type=jnp.float32)
    counts = jnp.sum(onehot, axis=1)
    pooled = sums / jnp.maximum(counts, 1.0)[:, None]
    out_ref[...] = (jnp.dot(pooled, linW_ref[...],
                            preferred_element_type=jnp.float32)
                    + linb_ref[...])


def _tc_final(part, pd0, pd1, selfc, selfd, b, batchp, lin_W, lin_b):
    return pl.pallas_call(
        _tc_final_body,
        out_shape=jax.ShapeDtypeStruct((G, OUT), jnp.float32),
    )(part, pd0, pd1, selfc, selfd, b, batchp, lin_W, lin_b)


def _sc_body(h_h, as_h, ad_h, src_h, dst_h, zden_h,
             part_h, partd0_h, partd1_h, *scr):
    a_s_v, a_d_v = scr[0], scr[1]
    rows = list(scr[2:4])
    srcr = list(scr[4:8])
    dstr = list(scr[8:12])
    wr = list(scr[12:16])
    acc_s, den_s = scr[16], scr[17]
    sg = list(scr[18:20])
    ss = list(scr[20:22])
    ssrc = list(scr[22:26])
    sdst = list(scr[26:30])
    sw = list(scr[30:34])
    st0, st1 = scr[34], scr[35]

    c = lax.axis_index("c")
    s = lax.axis_index("s")
    kq = jnp.where(c == 0, K0, K1)
    ebase = jnp.where(c == 0, s * (K0 * C), NS * (K0 * C) + s * (K1 * C))

    rsl = pl.ds(s * RPS, RPS)

    pltpu.async_copy(as_h, a_s_v, st0)
    pltpu.async_copy(ad_h, a_d_v, st1)

    @pl.when(s == 0)
    def _():
        pltpu.sync_copy(zden_h, den_s)

    zv = jnp.zeros((L,), jnp.float32)
    zi = jnp.zeros((L,), jnp.int32)
    for i in range(0, C):
        for j in range(H // L):
            rows[1][i, pl.ds(j * L, L)] = zv
    for t in range(RPS // C):
        pltpu.sync_copy(rows[1], acc_s.at[pl.ds(s * RPS + t * C, C)])
    rem = RPS % C
    if rem:
        pltpu.sync_copy(rows[1].at[pl.ds(0, rem)],
                        acc_s.at[pl.ds(s * RPS + (RPS // C) * C, rem)])

    pltpu.make_async_copy(as_h, a_s_v, st0).wait()
    pltpu.make_async_copy(ad_h, a_d_v, st1).wait()
    plsc.subcore_barrier()

    def _idx_copy(k, sl):
        return pltpu.make_async_copy(
            src_h.at[pl.ds(ebase + k * C, C)], srcr[sl], ssrc[sl])

    def _dst_copy(k, sl):
        return pltpu.make_async_copy(
            dst_h.at[pl.ds(ebase + k * C, C)], dstr[sl], sdst[sl])

    def _gather(sl, b):
        return pltpu.make_async_copy(h_h.at[srcr[sl]], rows[b], sg[b])

    def _scat_rows(b, sl):
        return pltpu.make_async_copy(rows[b], acc_s.at[dstr[sl]], ss[b])

    def _scat_w(sl):
        return pltpu.make_async_copy(wr[sl], den_s.at[dstr[sl]], sw[sl])

    for g in range(0, C, L):
        wr[3][pl.ds(g, L)] = zv
        dstr[3][pl.ds(g, L)] = zi
    pltpu.async_copy(rows[1], acc_s.at[dstr[3]], ss[1], add=True)
    pltpu.async_copy(wr[3], den_s.at[dstr[3]], sw[3], add=True)

    for j in range(3):
        _idx_copy(j, j).start()
        _dst_copy(j, j).start()
    _idx_copy(0, 0).wait()
    _gather(0, 0).start()

    @pl.loop(0, kq, step=4)
    def _phase4(kk):
        for p in range(4):
            k = kk + p
            b = p % 2

            _scat_rows(1 - b, (p - 1) % 4).wait()
            _scat_w((p - 1) % 4).wait()

            kn = jnp.minimum(k + 1, kq - 1)
            _idx_copy(kn, (p + 1) % 4).wait()
            _gather((p + 1) % 4, 1 - b).start()

            kp = jnp.minimum(k + 3, kq - 1)
            _idx_copy(kp, (p + 3) % 4).start()
            _dst_copy(kp, (p + 3) % 4).start()

            _gather(p % 4, b).wait()
            _dst_copy(k, p).wait()
            for g in range(0, C, L):
                si = srcr[p % 4][pl.ds(g, L)]
                di = dstr[p][pl.ds(g, L)]
                zz = (plsc.load_gather(a_s_v, [si])
                      + plsc.load_gather(a_d_v, [di]))
                wr[p][pl.ds(g, L)] = jnp.exp(_leaky(zz))

            @plsc.parallel_loop(0, C, unroll=4)
            def _row(i):
                wv = plsc.load_gather(wr[p], [jnp.full((L,), 0, jnp.int32) + i])
                for j in range(H // L):
                    sl = pl.ds(j * L, L)
                    rows[b][i, sl] = rows[b][i, sl] * wv

            pltpu.async_copy(rows[b], acc_s.at[dstr[p]], ss[b], add=True)
            pltpu.async_copy(wr[p], den_s.at[dstr[p]], sw[p], add=True)

    _scat_rows((K - 1) % 2, (K - 1) % 4).wait()
    _scat_w((K - 1) % 4).wait()
    _gather(0, K % 2).wait()
    _idx_copy(kq - 1, (K + 1) % 4).wait()
    _idx_copy(kq - 1, (K + 2) % 4).wait()
    _dst_copy(kq - 1, K % 4).wait()
    _dst_copy(kq - 1, (K + 1) % 4).wait()
    _dst_copy(kq - 1, (K + 2) % 4).wait()

    plsc.subcore_barrier()
    pltpu.sync_copy(acc_s.at[rsl], part_h.at[c, rsl])

    @pl.when(jnp.logical_and(s == 0, c == 0))
    def _():
        pltpu.sync_copy(den_s, partd0_h)

    @pl.when(jnp.logical_and(s == 0, c == 1))
    def _():
        pltpu.sync_copy(den_s, partd1_h)


def _sc_layer(h, a_s, a_d, srcp, dstp3, zden):
    mesh = plsc.VectorSubcoreMesh(core_axis_name="c", subcore_axis_name="s",
                                  num_cores=NC, num_subcores=NS)
    cp = pltpu.CompilerParams()
    if "needs_layout_passes" in pltpu.CompilerParams.__dataclass_fields__:
        cp = dataclasses.replace(cp, needs_layout_passes=False)
    kern = pl.kernel(
        _sc_body,
        out_type=(
            jax.ShapeDtypeStruct((NC, NP, H), jnp.float32),
            jax.ShapeDtypeStruct((NP,), jnp.float32),
            jax.ShapeDtypeStruct((NP,), jnp.float32),
        ),
        mesh=mesh,
        compiler_params=cp,
        scratch_types=(
            [
                pltpu.VMEM((NP,), jnp.float32),
                pltpu.VMEM((NP,), jnp.float32),
            ]
            + [pltpu.VMEM((C, H), jnp.float32)] * 2
            + [pltpu.VMEM((C,), jnp.int32)] * 8
            + [pltpu.VMEM((C,), jnp.float32)] * 4
            + [
                pltpu.VMEM_SHARED((NP, H), jnp.float32),
                pltpu.VMEM_SHARED((NP,), jnp.float32),
            ]
            + [pltpu.SemaphoreType.DMA] * 18
        ),
    )
    return kern(h, a_s, a_d, srcp, dstp3, zden)


def kernel(x, edge_index, batch, W1, a_src1, a_dst1, b1,
           W2, a_src2, a_dst2, b2, lin_W, lin_b):
    src = edge_index[0].astype(jnp.int32)
    dst = edge_index[1].astype(jnp.int32)
    pad_e = EP - E
    srcp = jnp.concatenate([src, jnp.zeros((pad_e,), jnp.int32)])
    pad_dst = N + jnp.arange(pad_e, dtype=jnp.int32) % (NP - N)
    dstp = jnp.concatenate([dst, pad_dst])
    xp = jnp.pad(x, ((0, NP - N), (0, 0)))
    batchp = jnp.concatenate(
        [batch.astype(jnp.int32), jnp.full((NP - N,), G, jnp.int32)])
    zden = jnp.zeros((NP,), jnp.float32)

    h1, selfc1, selfd1, a_s1, a_d1 = _tc_prep(xp, W1, a_src1, a_dst1)
    part1, pd1a, pd1b = _sc_layer(h1, a_s1, a_d1, srcp, dstp, zden)
    h2, selfc2, selfd2, a_s2, a_d2 = _tc_mid(
        part1, pd1a, pd1b, selfc1, selfd1, b1, W2, a_src2, a_dst2)
    part2, pd2a, pd2b = _sc_layer(h2, a_s2, a_d2, srcp, dstp, zden)
    return _tc_final(part2, pd2a, pd2b, selfc2, selfd2, b2, batchp,
                     lin_W, lin_b)
